# baseline (speedup 1.0000x reference)
import functools

import jax
import jax.numpy as jnp
from jax import lax
from jax.experimental import pallas as pl
from jax.experimental.pallas import tpu as pltpu
from jax.experimental.pallas import tpu_sc as plsc

D_MODEL = 1024
MAX_SEQ = 2048
BATCH = 16
N_ROWS = BATCH * MAX_SEQ
NUM_WORKERS = 32
CHUNK = 64
HALF = 32
ZROWS = 16
ZLAG = 6

_mesh = plsc.VectorSubcoreMesh(core_axis_name="c", subcore_axis_name="s")


@functools.partial(
    pl.kernel,
    mesh=_mesh,
    out_type=jax.ShapeDtypeStruct((N_ROWS, D_MODEL), jnp.float32),
    scratch_types=[
        pltpu.VMEM((16,), jnp.int32),
        pltpu.VMEM((HALF,), jnp.int32),
        pltpu.VMEM((ZROWS,), jnp.int32),
        pltpu.VMEM((CHUNK, D_MODEL), jnp.float32),
        pltpu.VMEM((ZROWS, D_MODEL), jnp.float32),
        pltpu.VMEM((ZROWS, D_MODEL), jnp.float32),
        pltpu.SemaphoreType.DMA,
        pltpu.SemaphoreType.DMA,
        pltpu.SemaphoreType.DMA,
    ],
)
def _pe_lookup(len_hbm, pe_hbm, pes_hbm, out_hbm, lens_v, idx_v, zidx_v,
               data_v, zero_v, mixb_v, gsem, csem, msem):
    cid = lax.axis_index("c")
    sid = lax.axis_index("s")
    wid = sid * 2 + cid
    t_lo = wid * CHUNK
    iota16 = lax.broadcasted_iota(jnp.int32, (16,), 0)

    pltpu.sync_copy(len_hbm, lens_v)
    l_all = lens_v[...]

    zvec = jnp.zeros((16,), jnp.int32)
    for j in range(ZROWS // 16):
        zidx_v[pl.ds(j * 16, 16)] = zvec
    d_stage = pltpu.make_async_copy(pes_hbm.at[pl.ds(t_lo, CHUNK)], data_v,
                                    gsem)
    d_zstage = pltpu.make_async_copy(pe_hbm.at[zidx_v], zero_v, gsem)
    d_stage.start()
    d_zstage.start()
    d_stage.wait()
    d_zstage.wait()

    metas = []
    for k in range(BATCH):
        l_k = l_all[k]
        halves = []
        for h in range(CHUNK // HALF):
            ht0 = t_lo + h * HALF
            row = k * MAX_SEQ + ht0
            halves.append(dict(
                ht0=ht0,
                copy=ht0 + HALF <= l_k,
                zero=ht0 >= l_k,
                mix=(ht0 < l_k) & (l_k < ht0 + HALF),
                d_s=pltpu.make_async_copy(
                    data_v.at[pl.ds(h * HALF, HALF)],
                    out_hbm.at[pl.ds(row, HALF)], csem),
                d_z=[pltpu.make_async_copy(
                    zero_v, out_hbm.at[pl.ds(row + z * ZROWS, ZROWS)], csem)
                    for z in range(HALF // ZROWS)],
            ))
        metas.append(dict(l_k=l_k, halves=halves))

    prev_mix = None

    for k in range(BATCH + ZLAG):
        if k < BATCH:
            m = metas[k]
            for hm in m["halves"]:
                @pl.when(hm["copy"])
                def _(hm=hm):
                    hm["d_s"].start()

                @pl.when(hm["zero"])
                def _(hm=hm):
                    for d in hm["d_z"]:
                        d.start()

            any_mix = m["halves"][0]["mix"] | m["halves"][1]["mix"]
            h_sel = jnp.where(m["halves"][1]["mix"], 1, 0)
            base = h_sel * HALF
            row_sel = k * MAX_SEQ + t_lo + base
            r_cut = m["l_k"] - (t_lo + base)
            g16 = r_cut >= ZROWS
            goff = jnp.where(g16, ZROWS, 0)
            c = r_cut - goff
            d_head = pltpu.make_async_copy(
                data_v.at[pl.ds(base, ZROWS)],
                out_hbm.at[pl.ds(row_sel, ZROWS)], msem)
            d_cut = pltpu.make_async_copy(
                mixb_v, out_hbm.at[pl.ds(row_sel + goff, ZROWS)], msem)
            d_tz = pltpu.make_async_copy(
                zero_v, out_hbm.at[pl.ds(row_sel + ZROWS, ZROWS)], msem)

            if prev_mix is not None:
                prev_mix.wait()

            @pl.when(any_mix)
            def _():
                zf = jnp.zeros((16,), jnp.float32)

                def body(i, carry):
                    r = i // (D_MODEL // 16)
                    cl = (i % (D_MODEL // 16)) * 16
                    src = data_v[base + goff + r, pl.ds(cl, 16)]
                    mixb_v[r, pl.ds(cl, 16)] = jnp.where(r < c, src, zf)
                    return carry

                lax.fori_loop(0, ZROWS * (D_MODEL // 16), body, jnp.int32(0))

            @pl.when(any_mix & g16)
            def _():
                d_head.start()

            @pl.when(any_mix)
            def _():
                d_cut.start()

            @pl.when(any_mix & jnp.logical_not(g16))
            def _():
                d_tz.start()

            prev_mix = _MixWait(any_mix, g16, d_head, d_cut, d_tz)

        if 0 <= k - ZLAG < BATCH:
            mz = metas[k - ZLAG]
            for hm in mz["halves"]:
                @pl.when(hm["copy"])
                def _(hm=hm):
                    hm["d_s"].wait()

                @pl.when(hm["zero"])
                def _(hm=hm):
                    for d in hm["d_z"]:
                        d.wait()

    if prev_mix is not None:
        prev_mix.wait()


class _MixWait:
    def __init__(self, any_mix, g16, d_head, d_cut, d_tz):
        self._args = (any_mix, g16, d_head, d_cut, d_tz)

    def wait(self):
        any_mix, g16, d_head, d_cut, d_tz = self._args

        @pl.when(any_mix & g16)
        def _():
            d_head.wait()

        @pl.when(any_mix & jnp.logical_not(g16))
        def _():
            d_tz.wait()

        @pl.when(any_mix)
        def _():
            d_cut.wait()


def kernel(input_len, position_encoding):
    out = _pe_lookup(input_len.astype(jnp.int32), position_encoding,
                     position_encoding[1:])
    return out.reshape(BATCH, MAX_SEQ, D_MODEL)
